# Initial kernel scaffold; baseline (speedup 1.0000x reference)
import jax
import jax.numpy as jnp
from jax.experimental import pallas as pl


def kernel(x, Wq, bq, keys, values):
    raise NotImplementedError("write your pallas kernel here")


# speedup vs baseline: 2.7377x; 2.7377x over previous
import functools

import jax
import jax.numpy as jnp
from jax import lax
from jax.experimental import pallas as pl
from jax.experimental.pallas import tpu as pltpu
from jax.experimental.pallas import tpu_sc as plsc

HEADS = 4
KNN = 16
NK = 512
HALF = 256
KD = 512
BS = 4096
VD = 512
HK = HEADS * KNN
BLK = 512
NEG = -1e30


def _topk16(s, n, payload=None):
    iota = lax.broadcasted_iota(jnp.int32, s.shape, 1)
    vals, idxs = [], []
    for _ in range(KNN):
        m = jnp.max(s, axis=1, keepdims=True)
        am = jnp.min(jnp.where(s == m, iota, n), axis=1, keepdims=True)
        if payload is None:
            idxs.append(am)
        else:
            idxs.append(jnp.min(jnp.where(iota == am, payload, jnp.int32(2**31 - 1)),
                                axis=1, keepdims=True))
        vals.append(m)
        s = jnp.where(iota == am, NEG, s)
    return jnp.concatenate(vals, axis=1), jnp.concatenate(idxs, axis=1)


def _tc_body(x_ref, wqt_ref, bq_ref, keys_ref, idx_ref, w_ref):
    x = x_ref[...]
    q = jnp.dot(x, wqt_ref[...], preferred_element_type=jnp.float32) + bq_ref[...]
    for h in range(HEADS):
        q1 = q[:, h * KD: h * KD + HALF]
        q2 = q[:, h * KD + HALF: (h + 1) * KD]
        k1 = keys_ref[2 * h * NK: (2 * h + 1) * NK, :]
        k2 = keys_ref[(2 * h + 1) * NK: (2 * h + 2) * NK, :]
        dn = (((1,), (1,)), ((), ()))
        s1 = lax.dot_general(q1, k1, dn, preferred_element_type=jnp.float32)
        s2 = lax.dot_general(q2, k2, dn, preferred_element_type=jnp.float32)
        v1, i1 = _topk16(s1, NK)
        v2, i2 = _topk16(s2, NK)
        all_s = jnp.concatenate([v1[:, a:a + 1] + v2 for a in range(KNN)], axis=1)
        all_i = jnp.concatenate([i1[:, a:a + 1] * NK + i2 for a in range(KNN)], axis=1)
        v, idx = _topk16(all_s, KNN * KNN, payload=all_i)
        w = jnp.exp(v - v[:, :1])
        w = w / jnp.sum(w, axis=1, keepdims=True)
        idx_ref[:, h * KNN:(h + 1) * KNN] = idx
        w_ref[:, h * KNN:(h + 1) * KNN] = w


def _tc_topk(x_flat, wqt, bq2, keys):
    return pl.pallas_call(
        _tc_body,
        grid=(BS // BLK,),
        in_specs=[
            pl.BlockSpec((BLK, 1024), lambda i: (i, 0)),
            pl.BlockSpec((1024, HEADS * KD), lambda i: (0, 0)),
            pl.BlockSpec((1, HEADS * KD), lambda i: (0, 0)),
            pl.BlockSpec((2 * HEADS * NK, HALF), lambda i: (0, 0)),
        ],
        out_specs=[
            pl.BlockSpec((BLK, HK), lambda i: (i, 0)),
            pl.BlockSpec((BLK, HK), lambda i: (i, 0)),
        ],
        out_shape=[
            jax.ShapeDtypeStruct((BS, HK), jnp.int32),
            jax.ShapeDtypeStruct((BS, HK), jnp.float32),
        ],
    )(x_flat, wqt, bq2, keys)


_NC, _NS = 2, 16
_NW = _NC * _NS
_TPW = BS // _NW
_NVS = VD // 16


@functools.cache
def _make_sc_bag():
    return functools.partial(
        pl.kernel,
        mesh=plsc.VectorSubcoreMesh(core_axis_name="c", subcore_axis_name="s"),
        compiler_params=pltpu.CompilerParams(needs_layout_passes=False),
        out_type=jax.ShapeDtypeStruct((BS, VD), jnp.float32),
        scratch_types=[
            pltpu.VMEM((_TPW, HK), jnp.int32),
            pltpu.VMEM((_TPW * HK,), jnp.float32),
            pltpu.VMEM((2, HK, VD), jnp.float32),
            pltpu.VMEM((2, VD), jnp.float32),
            pltpu.SemaphoreType.DMA,
            pltpu.SemaphoreType.DMA,
            pltpu.SemaphoreType.DMA,
            pltpu.SemaphoreType.DMA,
        ],
    )(_sc_bag_body)


def _sc_bag_body(values_hbm, idx_hbm, w_hbm, out_hbm,
                 idx_v, w_v, rows_v, acc_v, sg0, sg1, so0, so1):
    wid = lax.axis_index("s") * _NC + lax.axis_index("c")
    base = wid * _TPW
    pltpu.sync_copy(idx_hbm.at[pl.ds(base, _TPW)], idx_v)
    pltpu.sync_copy(w_hbm.at[pl.ds(base * HK, _TPW * HK)], w_v)
    sg = (sg0, sg1)
    so = (so0, so1)
    pltpu.async_copy(values_hbm.at[idx_v.at[0]], rows_v.at[0], sg0)
    pltpu.async_copy(values_hbm.at[idx_v.at[1]], rows_v.at[1], sg1)

    def body(i, carry):
        for b in (0, 1):
            t = 2 * i + b
            pltpu.make_async_copy(values_hbm.at[idx_v.at[t]], rows_v.at[b], sg[b]).wait()

            def jbody(j, acc):
                wb = plsc.load_gather(
                    w_v, [jnp.full((16,), t * HK + j, dtype=jnp.int32)])
                return tuple(acc[v] + rows_v[b, j, pl.ds(v * 16, 16)] * wb
                             for v in range(_NVS))

            acc = lax.fori_loop(
                0, HK, jbody,
                tuple(jnp.zeros((16,), jnp.float32) for _ in range(_NVS)))

            @pl.when(t >= 2)
            def _():
                pltpu.make_async_copy(acc_v.at[b], out_hbm.at[base + t - 2],
                                      so[b]).wait()

            for v in range(_NVS):
                acc_v[b, pl.ds(v * 16, 16)] = acc[v]
            pltpu.async_copy(acc_v.at[b], out_hbm.at[base + t], so[b])

            @pl.when(t + 2 < _TPW)
            def _():
                pltpu.async_copy(values_hbm.at[idx_v.at[t + 2]], rows_v.at[b], sg[b])
        return carry

    lax.fori_loop(0, _TPW // 2, body, 0)
    pltpu.make_async_copy(acc_v.at[0], out_hbm.at[base + _TPW - 2], so0).wait()
    pltpu.make_async_copy(acc_v.at[1], out_hbm.at[base + _TPW - 1], so1).wait()


def kernel(x, Wq, bq, keys, values):
    Bb, Tt, C = x.shape
    x_flat = x.reshape(-1, C)
    idx, w = _tc_topk(x_flat, Wq.T, bq.reshape(1, -1), keys)
    out = _make_sc_bag()(values, idx, w.reshape(-1))
    return out.reshape(Bb, Tt, VD)
